# baseline (speedup 1.0000x reference)
import functools

import jax
import jax.numpy as jnp
from jax import lax
from jax.experimental import pallas as pl
from jax.experimental.pallas import tpu as pltpu
from jax.experimental.pallas import tpu_sc as plsc

S = 200
B = 4096
MAX_SEQ = 200
D = 64
SB = S * B
NC = 2
NS = 16
NW = NC * NS
RW = SB // NW
C = 512
GSUB = 128
NCHUNK = RW // C
G = C // 16
EPS = 1e-5 / 64.0


def _rsqrt(v):
    i = lax.bitcast_convert_type(v, jnp.int32)
    i = jnp.int32(0x5F3759DF) - (i >> 1)
    y = lax.bitcast_convert_type(i, jnp.float32)
    h = v * jnp.float32(0.5)
    for _ in range(3):
        y = y * (jnp.float32(1.5) - h * y * y)
    return y


def _body(seq_hbm, pid_hbm, item_hbm, pos_hbm, w_hbm, b_hbm, out_hbm,
          idx_a, idx_p, buf_rows, buf_x, pos_v, w_v, b_v, sem):
    wid = lax.axis_index("c") * NS + lax.axis_index("s")
    base0 = wid * RW

    pltpu.sync_copy(pos_hbm, pos_v)
    pltpu.sync_copy(w_hbm, w_v)
    pltpu.sync_copy(b_hbm, b_v)

    def _scale(i, carry):
        sl = pl.ds(i * 16, 16)
        pos_v[sl] = pos_v[sl] * jnp.float32(0.125)
        return carry
    lax.fori_loop(0, (MAX_SEQ * D) // 16, _scale, 0)

    iota16 = lax.iota(jnp.int32, 16)
    w_regs = [w_v[pl.ds(k * 16, 16)] for k in range(4)]
    b_regs = [b_v[pl.ds(k * 16, 16)] for k in range(4)]
    zero_f = jnp.zeros((16,), jnp.float32)
    col0 = jnp.zeros((16,), jnp.int32)

    def _chunk(c, carry):
        base = base0 + c * C
        pltpu.sync_copy(seq_hbm.at[pl.ds(base, C)], idx_a)
        pltpu.sync_copy(pid_hbm.at[pl.ds(base, C)], idx_p)
        cps = [
            pltpu.async_copy(
                item_hbm.at[idx_a.at[pl.ds(j * GSUB, GSUB)]],
                buf_rows.at[pl.ds(j * GSUB, GSUB)],
                sem,
            )
            for j in range(C // GSUB)
        ]
        for cp in cps:
            cp.wait()

        def _group(g, carry):
            rvec = g * 16 + iota16
            pld = idx_p[pl.ds(g * 16, 16)]
            ip0 = pld * D
            ix0 = rvec * D

            acc1 = [zero_f, zero_f, zero_f, zero_f]
            acc2 = [zero_f, zero_f, zero_f, zero_f]
            for d in range(D):
                a = plsc.load_gather(buf_rows, [rvec, col0 + d])
                p = plsc.load_gather(pos_v, [ip0 + d])
                x = a + p
                plsc.store_scatter(buf_x, [ix0 + d], x)
                acc1[d % 4] = acc1[d % 4] + x
                acc2[d % 4] = acc2[d % 4] + x * x
            s1 = (acc1[0] + acc1[1]) + (acc1[2] + acc1[3])
            s2 = (acc2[0] + acc2[1]) + (acc2[2] + acc2[3])
            m = s1 * jnp.float32(1.0 / D)
            var = s2 * jnp.float32(1.0 / D) - m * m + jnp.float32(EPS)
            s = _rsqrt(var)
            u = m * s
            goff = g * (16 * D)
            for r in range(16):
                sr = s[r]
                ur = u[r]
                for k in range(4):
                    sl = pl.ds(goff + r * D + k * 16, 16)
                    x = buf_x[sl]
                    buf_x[sl] = (x * sr - ur) * w_regs[k] + b_regs[k]
            return carry

        lax.fori_loop(0, G, _group, 0)
        pltpu.sync_copy(buf_x, out_hbm.at[pl.ds(base * D, C * D)])
        return carry

    lax.fori_loop(0, NCHUNK, _chunk, 0)


@jax.jit
def _emb(seq_flat, pid_flat, item_table, pos_flat, ln_weight, ln_bias):
    mesh = plsc.VectorSubcoreMesh(core_axis_name="c", subcore_axis_name="s")
    f = functools.partial(
        pl.kernel,
        out_type=jax.ShapeDtypeStruct((SB * D,), jnp.float32),
        mesh=mesh,
        scratch_types=[
            pltpu.VMEM((C,), jnp.int32),
            pltpu.VMEM((C,), jnp.int32),
            pltpu.VMEM((C, D), jnp.float32),
            pltpu.VMEM((C * D,), jnp.float32),
            pltpu.VMEM((MAX_SEQ * D,), jnp.float32),
            pltpu.VMEM((D,), jnp.float32),
            pltpu.VMEM((D,), jnp.float32),
            pltpu.SemaphoreType.DMA,
        ],
        compiler_params=pltpu.CompilerParams(
            needs_layout_passes=False, use_tc_tiling_on_sc=False),
    )(_body)
    return f(seq_flat, pid_flat, item_table, pos_flat, ln_weight, ln_bias)


def kernel(input_sequence, position_ids, item_table, pos_table, ln_weight, ln_bias):
    seq_flat = input_sequence.reshape(SB)
    pid_flat = position_ids.reshape(SB)
    pos_flat = pos_table.reshape(MAX_SEQ * D)
    out = _emb(seq_flat, pid_flat, item_table, pos_flat, ln_weight, ln_bias)
    return out.reshape(S, B, D)


# speedup vs baseline: 1.4194x; 1.4194x over previous
import functools

import jax
import jax.numpy as jnp
from jax import lax
from jax.experimental import pallas as pl
from jax.experimental.pallas import tpu as pltpu
from jax.experimental.pallas import tpu_sc as plsc

S = 200
B = 4096
MAX_SEQ = 200
D = 64
SB = S * B
NC = 2
NS = 16
NW = NC * NS
RW = SB // NW
C = 512
GSUB = 128
NCHUNK = RW // C
G = C // 16
EPS = 1e-5 / 64.0


def _rsqrt(v):
    i = lax.bitcast_convert_type(v, jnp.int32)
    i = jnp.int32(0x5F3759DF) - (i >> 1)
    y = lax.bitcast_convert_type(i, jnp.float32)
    h = v * jnp.float32(0.5)
    for _ in range(3):
        y = y * (jnp.float32(1.5) - h * y * y)
    return y


def _body(seq_hbm, pid_hbm, item_hbm, pos_hbm, w_hbm, b_hbm, out_hbm,
          idx_a, idx_p, buf_rows, buf_x, pos_v, w_v, b_v, sem):
    wid = lax.axis_index("c") * NS + lax.axis_index("s")
    base0 = wid * RW

    pltpu.sync_copy(pos_hbm, pos_v)
    pltpu.sync_copy(w_hbm, w_v)
    pltpu.sync_copy(b_hbm, b_v)

    def _scale(i, carry):
        sl = pl.ds(i * 16, 16)
        pos_v[sl] = pos_v[sl] * jnp.float32(0.125)
        return carry
    lax.fori_loop(0, (MAX_SEQ * D) // 16, _scale, 0)

    iota16 = lax.iota(jnp.int32, 16)
    w_regs = [w_v[pl.ds(k * 16, 16)] for k in range(4)]
    b_regs = [b_v[pl.ds(k * 16, 16)] for k in range(4)]
    zero_f = jnp.zeros((16,), jnp.float32)
    col0 = jnp.zeros((16,), jnp.int32)

    def _chunk(c, carry):
        base = base0 + c * C
        pltpu.sync_copy(seq_hbm.at[pl.ds(base, C)], idx_a)
        pltpu.sync_copy(pid_hbm.at[pl.ds(base, C)], idx_p)
        cps = [
            pltpu.async_copy(
                item_hbm.at[idx_a.at[pl.ds(j * GSUB, GSUB)]],
                buf_rows.at[pl.ds(j * GSUB, GSUB)],
                sem,
            )
            for j in range(C // GSUB)
        ]
        for cp in cps:
            cp.wait()

        def _group(g, carry):
            rvec = g * 16 + iota16
            pld = idx_p[pl.ds(g * 16, 16)]
            ip0 = pld * D
            ix0 = rvec * D

            U = 8

            def _p1(blk, st):
                colb, ipb, ixb, a1, b1, a2, b2 = st
                xs = []
                for d in range(U):
                    a = plsc.load_gather(buf_rows, [rvec, colb + d])
                    p = plsc.load_gather(pos_v, [ipb + d])
                    xs.append(a + p)
                for d in range(U):
                    x = xs[d]
                    plsc.store_scatter(buf_x, [ixb + d], x)
                    if d % 2:
                        b1 = b1 + x
                        b2 = b2 + x * x
                    else:
                        a1 = a1 + x
                        a2 = a2 + x * x
                return (colb + U, ipb + U, ixb + U, a1, b1, a2, b2)

            _, _, _, a1, b1, a2, b2 = lax.fori_loop(
                0, D // U, _p1,
                (col0, ip0, ix0, zero_f, zero_f, zero_f, zero_f))
            s1 = a1 + b1
            s2 = a2 + b2
            m = s1 * jnp.float32(1.0 / D)
            var = s2 * jnp.float32(1.0 / D) - m * m + jnp.float32(EPS)
            s = _rsqrt(var)
            u = m * s
            goff = g * (16 * D)
            for r in range(16):
                sr = s[r]
                ur = u[r]
                for k in range(4):
                    sl = pl.ds(goff + r * D + k * 16, 16)
                    x = buf_x[sl]
                    buf_x[sl] = (x * sr - ur) * w_regs[k] + b_regs[k]
            return carry

        lax.fori_loop(0, G, _group, 0)
        pltpu.sync_copy(buf_x, out_hbm.at[pl.ds(base * D, C * D)])
        return carry

    lax.fori_loop(0, NCHUNK, _chunk, 0)


@jax.jit
def _emb(seq_flat, pid_flat, item_table, pos_flat, ln_weight, ln_bias):
    mesh = plsc.VectorSubcoreMesh(core_axis_name="c", subcore_axis_name="s")
    f = functools.partial(
        pl.kernel,
        out_type=jax.ShapeDtypeStruct((SB * D,), jnp.float32),
        mesh=mesh,
        scratch_types=[
            pltpu.VMEM((C,), jnp.int32),
            pltpu.VMEM((C,), jnp.int32),
            pltpu.VMEM((C, D), jnp.float32),
            pltpu.VMEM((C * D,), jnp.float32),
            pltpu.VMEM((MAX_SEQ * D,), jnp.float32),
            pltpu.VMEM((D,), jnp.float32),
            pltpu.VMEM((D,), jnp.float32),
            pltpu.SemaphoreType.DMA,
        ],
        compiler_params=pltpu.CompilerParams(
            needs_layout_passes=False, use_tc_tiling_on_sc=False),
    )(_body)
    return f(seq_flat, pid_flat, item_table, pos_flat, ln_weight, ln_bias)


def kernel(input_sequence, position_ids, item_table, pos_table, ln_weight, ln_bias):
    seq_flat = input_sequence.reshape(SB)
    pid_flat = position_ids.reshape(SB)
    pos_flat = pos_table.reshape(MAX_SEQ * D)
    out = _emb(seq_flat, pid_flat, item_table, pos_flat, ln_weight, ln_bias)
    return out.reshape(S, B, D)


# speedup vs baseline: 1.9525x; 1.3756x over previous
import functools

import jax
import jax.numpy as jnp
from jax import lax
from jax.experimental import pallas as pl
from jax.experimental.pallas import tpu as pltpu
from jax.experimental.pallas import tpu_sc as plsc

S = 200
B = 4096
MAX_SEQ = 200
D = 64
SB = S * B
NC = 2
NS = 16
NW = NC * NS
RW = SB // NW
C = 512
GSUB = 128
NCHUNK = RW // C
G = C // 16
EPS = 1e-5 / 64.0


def _rsqrt(v):
    i = lax.bitcast_convert_type(v, jnp.int32)
    i = jnp.int32(0x5F3759DF) - (i >> 1)
    y = lax.bitcast_convert_type(i, jnp.float32)
    h = v * jnp.float32(0.5)
    for _ in range(3):
        y = y * (jnp.float32(1.5) - h * y * y)
    return y


def _body(seq_hbm, pid_hbm, item_hbm, pos_hbm, w_hbm, b_hbm, out_hbm,
          idx_a, idx_p, buf_rows, buf_x, pos_v, w_v, b_v, scr, su, sem):
    wid = lax.axis_index("c") * NS + lax.axis_index("s")
    base0 = wid * RW

    pltpu.sync_copy(pos_hbm, pos_v)
    pltpu.sync_copy(w_hbm, w_v)
    pltpu.sync_copy(b_hbm, b_v)

    def _scale(i, carry):
        sl = pl.ds(i * 16, 16)
        pos_v[sl] = pos_v[sl] * jnp.float32(0.125)
        return carry
    lax.fori_loop(0, (MAX_SEQ * D) // 16, _scale, 0)

    iota17 = lax.iota(jnp.int32, 16) * 17
    w_regs = [w_v[pl.ds(k * 16, 16)] for k in range(4)]
    b_regs = [b_v[pl.ds(k * 16, 16)] for k in range(4)]
    zero_f = jnp.zeros((16,), jnp.float32)

    def _chunk(c, carry):
        base = base0 + c * C
        pltpu.sync_copy(seq_hbm.at[pl.ds(base, C)], idx_a)
        pltpu.sync_copy(pid_hbm.at[pl.ds(base, C)], idx_p.at[pl.ds(0, C)])
        cps = [
            pltpu.async_copy(
                item_hbm.at[idx_a.at[pl.ds(j * GSUB, GSUB)]],
                buf_rows.at[pl.ds(j * GSUB, GSUB)],
                sem,
            )
            for j in range(C // GSUB)
        ]
        for cp in cps:
            cp.wait()

        def _group(g, carry):
            grow = g * 16
            gx = grow * D

            def _p1(rb, st):
                for rr in range(4):
                    r = rb * 4 + rr
                    row = grow + r
                    pb = idx_p[pl.ds(row, 16)][0] * D
                    xoff = gx + r * D
                    xs = []
                    for k in range(4):
                        a = buf_rows[row, pl.ds(k * 16, 16)]
                        p = pos_v[pl.ds(pb + k * 16, 16)]
                        x = a + p
                        buf_x[pl.ds(xoff + k * 16, 16)] = x
                        xs.append(x)
                    pr = (xs[0] + xs[1]) + (xs[2] + xs[3])
                    q0, q1, q2, q3 = (x * x for x in xs)
                    qr = (q0 + q1) + (q2 + q3)
                    scr[pl.ds(r * 17, 16)] = pr
                    scr[pl.ds((16 + r) * 17, 16)] = qr
                return st

            lax.fori_loop(0, 4, _p1, 0)

            s1a = s1b = s2a = s2b = zero_f
            for j in range(16):
                c1 = plsc.load_gather(scr, [iota17 + j])
                c2 = plsc.load_gather(scr, [iota17 + (16 * 17 + j)])
                if j % 2:
                    s1b = s1b + c1
                    s2b = s2b + c2
                else:
                    s1a = s1a + c1
                    s2a = s2a + c2
            s1 = s1a + s1b
            s2 = s2a + s2b
            m = s1 * jnp.float32(1.0 / D)
            var = s2 * jnp.float32(1.0 / D) - m * m + jnp.float32(EPS)
            s = _rsqrt(var)
            u = m * s
            su[pl.ds(0, 16)] = s
            su[pl.ds(16, 16)] = u

            def _p2(rb, st):
                for rr in range(4):
                    r = rb * 4 + rr
                    sr = su[pl.ds(r, 16)][0]
                    ur = su[pl.ds(16 + r, 16)][0]
                    xoff = gx + r * D
                    for k in range(4):
                        sl = pl.ds(xoff + k * 16, 16)
                        x = buf_x[sl]
                        buf_x[sl] = (x * sr - ur) * w_regs[k] + b_regs[k]
                return st

            lax.fori_loop(0, 4, _p2, 0)
            return carry

        lax.fori_loop(0, G, _group, 0)
        pltpu.sync_copy(buf_x, out_hbm.at[pl.ds(base * D, C * D)])
        return carry

    lax.fori_loop(0, NCHUNK, _chunk, 0)


@jax.jit
def _emb(seq_flat, pid_flat, item_table, pos_flat, ln_weight, ln_bias):
    mesh = plsc.VectorSubcoreMesh(core_axis_name="c", subcore_axis_name="s")
    f = functools.partial(
        pl.kernel,
        out_type=jax.ShapeDtypeStruct((SB * D,), jnp.float32),
        mesh=mesh,
        scratch_types=[
            pltpu.VMEM((C,), jnp.int32),
            pltpu.VMEM((C + 16,), jnp.int32),
            pltpu.VMEM((C, D), jnp.float32),
            pltpu.VMEM((C * D,), jnp.float32),
            pltpu.VMEM((MAX_SEQ * D,), jnp.float32),
            pltpu.VMEM((D,), jnp.float32),
            pltpu.VMEM((D,), jnp.float32),
            pltpu.VMEM((32 * 17,), jnp.float32),
            pltpu.VMEM((48,), jnp.float32),
            pltpu.SemaphoreType.DMA,
        ],
        compiler_params=pltpu.CompilerParams(
            needs_layout_passes=False, use_tc_tiling_on_sc=False),
    )(_body)
    return f(seq_flat, pid_flat, item_table, pos_flat, ln_weight, ln_bias)


def kernel(input_sequence, position_ids, item_table, pos_table, ln_weight, ln_bias):
    seq_flat = input_sequence.reshape(SB)
    pid_flat = position_ids.reshape(SB)
    pos_flat = pos_table.reshape(MAX_SEQ * D)
    out = _emb(seq_flat, pid_flat, item_table, pos_flat, ln_weight, ln_bias)
    return out.reshape(S, B, D)


# speedup vs baseline: 2.4347x; 1.2469x over previous
import functools

import jax
import jax.numpy as jnp
from jax import lax
from jax.experimental import pallas as pl
from jax.experimental.pallas import tpu as pltpu
from jax.experimental.pallas import tpu_sc as plsc

S = 200
B = 4096
MAX_SEQ = 200
D = 64
SB = S * B
NC = 2
NS = 16
NW = NC * NS
RW = SB // NW
C = 512
GSUB = 128
NCHUNK = RW // C
G = C // 16
EPS = 1e-5 / 64.0


def _splat(v, r):
    idx = jnp.full((16, 1), r, jnp.int32)
    dnums = lax.GatherDimensionNumbers(
        offset_dims=(), collapsed_slice_dims=(0,), start_index_map=(0,))
    return lax.gather(v, idx, dnums, (1,),
                      mode=lax.GatherScatterMode.PROMISE_IN_BOUNDS)


def _rsqrt(v):
    i = lax.bitcast_convert_type(v, jnp.int32)
    i = jnp.int32(0x5F3759DF) - (i >> 1)
    y = lax.bitcast_convert_type(i, jnp.float32)
    h = v * jnp.float32(0.5)
    for _ in range(3):
        y = y * (jnp.float32(1.5) - h * y * y)
    return y


def _body(seq_hbm, pid_hbm, item_hbm, pos_hbm, w_hbm, b_hbm, out_hbm,
          idx_a, idx_p, buf_rows, buf_x, pos_v, w_v, b_v, scr, sem):
    wid = lax.axis_index("c") * NS + lax.axis_index("s")
    base0 = wid * RW

    pltpu.sync_copy(pos_hbm, pos_v)
    pltpu.sync_copy(w_hbm, w_v)
    pltpu.sync_copy(b_hbm, b_v)

    def _scale(i, carry):
        sl = pl.ds(i * 16, 16)
        pos_v[sl] = pos_v[sl] * jnp.float32(0.125)
        return carry
    lax.fori_loop(0, (MAX_SEQ * D) // 16, _scale, 0)

    iota16 = lax.iota(jnp.int32, 16)
    iota17 = iota16 * 17
    iotak = [iota16 + k * 16 for k in range(4)]
    w_regs = [w_v[pl.ds(k * 16, 16)] for k in range(4)]
    b_regs = [b_v[pl.ds(k * 16, 16)] for k in range(4)]
    zero_f = jnp.zeros((16,), jnp.float32)

    def _chunk(c, carry):
        base = base0 + c * C
        pltpu.sync_copy(seq_hbm.at[pl.ds(base, C)], idx_a)
        pltpu.sync_copy(pid_hbm.at[pl.ds(base, C)], idx_p.at[pl.ds(0, C)])
        cps = [
            pltpu.async_copy(
                item_hbm.at[idx_a.at[pl.ds(j * GSUB, GSUB)]],
                buf_rows.at[pl.ds(j * GSUB, GSUB)],
                sem,
            )
            for j in range(C // GSUB)
        ]
        for cp in cps:
            cp.wait()

        def _group(g, carry):
            grow = g * 16
            gx = grow * D
            pldv = idx_p[pl.ds(grow, 16)]
            pb_all = pldv * D

            for r in range(16):
                row = grow + r
                pbv = _splat(pb_all, r)
                xoff = gx + r * D
                xs = []
                for k in range(4):
                    a = buf_rows[row, pl.ds(k * 16, 16)]
                    p = plsc.load_gather(pos_v, [pbv + iotak[k]])
                    x = a + p
                    buf_x[pl.ds(xoff + k * 16, 16)] = x
                    xs.append(x)
                pr = (xs[0] + xs[1]) + (xs[2] + xs[3])
                q0, q1, q2, q3 = (x * x for x in xs)
                qr = (q0 + q1) + (q2 + q3)
                scr[pl.ds(r * 17, 16)] = pr
                scr[pl.ds((16 + r) * 17, 16)] = qr

            s1a = s1b = s2a = s2b = zero_f
            for j in range(16):
                c1 = plsc.load_gather(scr, [iota17 + j])
                c2 = plsc.load_gather(scr, [iota17 + (16 * 17 + j)])
                if j % 2:
                    s1b = s1b + c1
                    s2b = s2b + c2
                else:
                    s1a = s1a + c1
                    s2a = s2a + c2
            s1 = s1a + s1b
            s2 = s2a + s2b
            m = s1 * jnp.float32(1.0 / D)
            var = s2 * jnp.float32(1.0 / D) - m * m + jnp.float32(EPS)
            s = _rsqrt(var)
            u = m * s

            for r in range(16):
                srv = _splat(s, r)
                urv = _splat(u, r)
                xoff = gx + r * D
                for k in range(4):
                    sl = pl.ds(xoff + k * 16, 16)
                    x = buf_x[sl]
                    buf_x[sl] = (x * srv - urv) * w_regs[k] + b_regs[k]
            return carry

        lax.fori_loop(0, G, _group, 0)
        pltpu.sync_copy(buf_x, out_hbm.at[pl.ds(base * D, C * D)])
        return carry

    lax.fori_loop(0, NCHUNK, _chunk, 0)


@jax.jit
def _emb(seq_flat, pid_flat, item_table, pos_flat, ln_weight, ln_bias):
    mesh = plsc.VectorSubcoreMesh(core_axis_name="c", subcore_axis_name="s")
    f = functools.partial(
        pl.kernel,
        out_type=jax.ShapeDtypeStruct((SB * D,), jnp.float32),
        mesh=mesh,
        scratch_types=[
            pltpu.VMEM((C,), jnp.int32),
            pltpu.VMEM((C + 16,), jnp.int32),
            pltpu.VMEM((C, D), jnp.float32),
            pltpu.VMEM((C * D,), jnp.float32),
            pltpu.VMEM((MAX_SEQ * D,), jnp.float32),
            pltpu.VMEM((D,), jnp.float32),
            pltpu.VMEM((D,), jnp.float32),
            pltpu.VMEM((32 * 17,), jnp.float32),
            pltpu.SemaphoreType.DMA,
        ],
        compiler_params=pltpu.CompilerParams(
            needs_layout_passes=False, use_tc_tiling_on_sc=False),
    )(_body)
    return f(seq_flat, pid_flat, item_table, pos_flat, ln_weight, ln_bias)


def kernel(input_sequence, position_ids, item_table, pos_table, ln_weight, ln_bias):
    seq_flat = input_sequence.reshape(SB)
    pid_flat = position_ids.reshape(SB)
    pos_flat = pos_table.reshape(MAX_SEQ * D)
    out = _emb(seq_flat, pid_flat, item_table, pos_flat, ln_weight, ln_bias)
    return out.reshape(S, B, D)


# speedup vs baseline: 2.4711x; 1.0150x over previous
import functools

import jax
import jax.numpy as jnp
from jax import lax
from jax.experimental import pallas as pl
from jax.experimental.pallas import tpu as pltpu
from jax.experimental.pallas import tpu_sc as plsc

S = 200
B = 4096
MAX_SEQ = 200
D = 64
SB = S * B
NC = 2
NS = 16
NW = NC * NS
RW = SB // NW
C = 256
GSUB = 128
NCHUNK = RW // C
G = C // 16
EPS = 1e-5 / 64.0


def _splat(v, r):
    idx = jnp.full((16, 1), r, jnp.int32)
    dnums = lax.GatherDimensionNumbers(
        offset_dims=(), collapsed_slice_dims=(0,), start_index_map=(0,))
    return lax.gather(v, idx, dnums, (1,),
                      mode=lax.GatherScatterMode.PROMISE_IN_BOUNDS)


def _rsqrt(v):
    i = lax.bitcast_convert_type(v, jnp.int32)
    i = jnp.int32(0x5F3759DF) - (i >> 1)
    y = lax.bitcast_convert_type(i, jnp.float32)
    h = v * jnp.float32(0.5)
    for _ in range(3):
        y = y * (jnp.float32(1.5) - h * y * y)
    return y


def _body(seq_hbm, pid_hbm, item_hbm, pos_hbm, w_hbm, b_hbm, out_hbm,
          idx_a0, idx_a1, idx_p0, idx_p1, rows0, rows1, bufx0, bufx1,
          pos_v, w_v, b_v, scr,
          sem_g0, sem_g1, sem_w0, sem_w1):
    wid = lax.axis_index("c") * NS + lax.axis_index("s")
    base0 = wid * RW

    pltpu.sync_copy(pos_hbm, pos_v)
    pltpu.sync_copy(w_hbm, w_v)
    pltpu.sync_copy(b_hbm, b_v)

    def _scale(i, carry):
        sl = pl.ds(i * 16, 16)
        pos_v[sl] = pos_v[sl] * jnp.float32(0.125)
        return carry
    lax.fori_loop(0, (MAX_SEQ * D) // 16, _scale, 0)

    iota16 = lax.iota(jnp.int32, 16)
    iota17 = iota16 * 17
    iotak = [iota16 + k * 16 for k in range(4)]
    w_regs = [w_v[pl.ds(k * 16, 16)] for k in range(4)]
    b_regs = [b_v[pl.ds(k * 16, 16)] for k in range(4)]
    zero_f = jnp.zeros((16,), jnp.float32)

    def _compute(rows_b, x_b, idxp_b):
        def _group(g, carry):
            grow = g * 16
            gx = grow * D
            pldv = idxp_b[pl.ds(grow, 16)]
            pb_all = pldv * D

            for r in range(16):
                row = grow + r
                pbv = _splat(pb_all, r)
                xoff = gx + r * D
                xs = []
                for k in range(4):
                    a = rows_b[row, pl.ds(k * 16, 16)]
                    p = plsc.load_gather(pos_v, [pbv + iotak[k]])
                    x = a + p
                    x_b[pl.ds(xoff + k * 16, 16)] = x
                    xs.append(x)
                pr = (xs[0] + xs[1]) + (xs[2] + xs[3])
                q0, q1, q2, q3 = (x * x for x in xs)
                qr = (q0 + q1) + (q2 + q3)
                scr[pl.ds(r * 17, 16)] = pr
                scr[pl.ds((16 + r) * 17, 16)] = qr

            s1a = s1b = s2a = s2b = zero_f
            for j in range(16):
                c1 = plsc.load_gather(scr, [iota17 + j])
                c2 = plsc.load_gather(scr, [iota17 + (16 * 17 + j)])
                if j % 2:
                    s1b = s1b + c1
                    s2b = s2b + c2
                else:
                    s1a = s1a + c1
                    s2a = s2a + c2
            s1 = s1a + s1b
            s2 = s2a + s2b
            m = s1 * jnp.float32(1.0 / D)
            var = s2 * jnp.float32(1.0 / D) - m * m + jnp.float32(EPS)
            sc = _rsqrt(var)
            u = m * sc

            for r in range(16):
                srv = _splat(sc, r)
                urv = _splat(u, r)
                xoff = gx + r * D
                for k in range(4):
                    sl = pl.ds(xoff + k * 16, 16)
                    x = x_b[sl]
                    x_b[sl] = (x * srv - urv) * w_regs[k] + b_regs[k]
            return carry

        lax.fori_loop(0, G, _group, 0)

    idx_as = (idx_a0, idx_a1)
    idx_ps = (idx_p0, idx_p1)
    rows = (rows0, rows1)
    bufx = (bufx0, bufx1)
    sem_g = (sem_g0, sem_g1)
    sem_w = (sem_w0, sem_w1)

    def _load_idx(n, P):
        base = base0 + n * C
        pltpu.sync_copy(seq_hbm.at[pl.ds(base, C)], idx_as[P])
        pltpu.sync_copy(pid_hbm.at[pl.ds(base, C)],
                        idx_ps[P].at[pl.ds(0, C)])

    def _gather_cps(P):
        return [
            pltpu.make_async_copy(
                item_hbm.at[idx_as[P].at[pl.ds(j * GSUB, GSUB)]],
                rows[P].at[pl.ds(j * GSUB, GSUB)],
                sem_g[P],
            )
            for j in range(C // GSUB)
        ]

    def _wb_cp(n, P):
        base = base0 + n * C
        return pltpu.make_async_copy(
            bufx[P], out_hbm.at[pl.ds(base * D, C * D)], sem_w[P])

    _load_idx(0, 0)
    for cp in _gather_cps(0):
        cp.start()

    def _step(ii, carry):
        for half in range(2):
            n = ii * 2 + half
            P = half
            for cp in _gather_cps(P):
                cp.wait()

            if half == 0:
                _load_idx(n + 1, 1 - P)
                for cp in _gather_cps(1 - P):
                    cp.start()
            else:
                @pl.when(ii < (NCHUNK // 2) - 1)
                def _prefetch():
                    _load_idx(n + 1, 1 - P)
                    for cp in _gather_cps(1 - P):
                        cp.start()

            @pl.when(ii > 0)
            def _drain():
                _wb_cp(n, P).wait()

            _compute(rows[P], bufx[P], idx_ps[P])
            _wb_cp(n, P).start()
        return carry

    lax.fori_loop(0, NCHUNK // 2, _step, 0)
    _wb_cp(NCHUNK - 2, 0).wait()
    _wb_cp(NCHUNK - 1, 1).wait()


@jax.jit
def _emb(seq_flat, pid_flat, item_table, pos_flat, ln_weight, ln_bias):
    mesh = plsc.VectorSubcoreMesh(core_axis_name="c", subcore_axis_name="s")
    f = functools.partial(
        pl.kernel,
        out_type=jax.ShapeDtypeStruct((SB * D,), jnp.float32),
        mesh=mesh,
        scratch_types=[
            pltpu.VMEM((C,), jnp.int32),
            pltpu.VMEM((C,), jnp.int32),
            pltpu.VMEM((C + 16,), jnp.int32),
            pltpu.VMEM((C + 16,), jnp.int32),
            pltpu.VMEM((C, D), jnp.float32),
            pltpu.VMEM((C, D), jnp.float32),
            pltpu.VMEM((C * D,), jnp.float32),
            pltpu.VMEM((C * D,), jnp.float32),
            pltpu.VMEM((MAX_SEQ * D,), jnp.float32),
            pltpu.VMEM((D,), jnp.float32),
            pltpu.VMEM((D,), jnp.float32),
            pltpu.VMEM((32 * 17,), jnp.float32),
            pltpu.SemaphoreType.DMA,
            pltpu.SemaphoreType.DMA,
            pltpu.SemaphoreType.DMA,
            pltpu.SemaphoreType.DMA,
        ],
        compiler_params=pltpu.CompilerParams(
            needs_layout_passes=False, use_tc_tiling_on_sc=False),
    )(_body)
    return f(seq_flat, pid_flat, item_table, pos_flat, ln_weight, ln_bias)


def kernel(input_sequence, position_ids, item_table, pos_table, ln_weight, ln_bias):
    seq_flat = input_sequence.reshape(SB)
    pid_flat = position_ids.reshape(SB)
    pos_flat = pos_table.reshape(MAX_SEQ * D)
    out = _emb(seq_flat, pid_flat, item_table, pos_flat, ln_weight, ln_bias)
    return out.reshape(S, B, D)


# speedup vs baseline: 2.8145x; 1.1390x over previous
import functools

import jax
import jax.numpy as jnp
from jax import lax
from jax.experimental import pallas as pl
from jax.experimental.pallas import tpu as pltpu
from jax.experimental.pallas import tpu_sc as plsc

S = 200
B = 4096
MAX_SEQ = 200
D = 64
SB = S * B
NC = 2
NS = 16
NW = NC * NS
RW = SB // NW
C = 256
GSUB = 128
NCHUNK = RW // C
G = C // 16
EPS = 1e-5


def _splat(v, r):
    idx = jnp.full((16, 1), r, jnp.int32)
    dnums = lax.GatherDimensionNumbers(
        offset_dims=(), collapsed_slice_dims=(0,), start_index_map=(0,))
    return lax.gather(v, idx, dnums, (1,),
                      mode=lax.GatherScatterMode.PROMISE_IN_BOUNDS)


def _rsqrt(v):
    i = lax.bitcast_convert_type(v, jnp.int32)
    i = jnp.int32(0x5F3759DF) - (i >> 1)
    y = lax.bitcast_convert_type(i, jnp.float32)
    h = v * jnp.float32(0.5)
    for _ in range(3):
        y = y * (jnp.float32(1.5) - h * y * y)
    return y


def _body(seq_hbm, pid_hbm, item_hbm, pos_hbm, w_hbm, b_hbm, out_hbm,
          idx_a0, idx_a1, idx_p0, idx_p1, rows0, rows1, prow0, prow1,
          bufx0, bufx1, w_v, b_v, scr,
          sem_g0, sem_g1, sem_w0, sem_w1):
    wid = lax.axis_index("c") * NS + lax.axis_index("s")
    base0 = wid * RW

    pltpu.sync_copy(w_hbm, w_v)
    pltpu.sync_copy(b_hbm, b_v)

    iota17 = lax.iota(jnp.int32, 16) * 17
    w_regs = [w_v[pl.ds(k * 16, 16)] for k in range(4)]
    b_regs = [b_v[pl.ds(k * 16, 16)] for k in range(4)]
    zero_f = jnp.zeros((16,), jnp.float32)

    def _compute(rows_b, pos_b, x_b):
        def _group(g, carry):
            grow = g * 16
            gx = grow * D

            for r in range(16):
                row = grow + r
                xoff = gx + r * D
                xs = []
                for k in range(4):
                    a = rows_b[row, pl.ds(k * 16, 16)]
                    p = pos_b[row, pl.ds(k * 16, 16)]
                    x = a * jnp.float32(8.0) + p
                    x_b[pl.ds(xoff + k * 16, 16)] = x
                    xs.append(x)
                pr = (xs[0] + xs[1]) + (xs[2] + xs[3])
                q0, q1, q2, q3 = (x * x for x in xs)
                qr = (q0 + q1) + (q2 + q3)
                scr[pl.ds(r * 17, 16)] = pr
                scr[pl.ds((16 + r) * 17, 16)] = qr

            s1a = s1b = s2a = s2b = zero_f
            for j in range(16):
                c1 = plsc.load_gather(scr, [iota17 + j])
                c2 = plsc.load_gather(scr, [iota17 + (16 * 17 + j)])
                if j % 2:
                    s1b = s1b + c1
                    s2b = s2b + c2
                else:
                    s1a = s1a + c1
                    s2a = s2a + c2
            s1 = s1a + s1b
            s2 = s2a + s2b
            m = s1 * jnp.float32(1.0 / D)
            var = s2 * jnp.float32(1.0 / D) - m * m + jnp.float32(EPS)
            sc = _rsqrt(var)
            u = m * sc

            for r in range(16):
                srv = _splat(sc, r)
                urv = _splat(u, r)
                xoff = gx + r * D
                for k in range(4):
                    sl = pl.ds(xoff + k * 16, 16)
                    x = x_b[sl]
                    x_b[sl] = (x * srv - urv) * w_regs[k] + b_regs[k]
            return carry

        lax.fori_loop(0, G, _group, 0)

    idx_as = (idx_a0, idx_a1)
    idx_ps = (idx_p0, idx_p1)
    rows = (rows0, rows1)
    prows = (prow0, prow1)
    bufx = (bufx0, bufx1)
    sem_g = (sem_g0, sem_g1)
    sem_w = (sem_w0, sem_w1)

    def _load_idx(n, P):
        base = base0 + n * C
        pltpu.sync_copy(seq_hbm.at[pl.ds(base, C)], idx_as[P])
        pltpu.sync_copy(pid_hbm.at[pl.ds(base, C)], idx_ps[P])

    def _gather_cps(P):
        cps = [
            pltpu.make_async_copy(
                item_hbm.at[idx_as[P].at[pl.ds(j * GSUB, GSUB)]],
                rows[P].at[pl.ds(j * GSUB, GSUB)],
                sem_g[P],
            )
            for j in range(C // GSUB)
        ]
        cps += [
            pltpu.make_async_copy(
                pos_hbm.at[idx_ps[P].at[pl.ds(j * GSUB, GSUB)]],
                prows[P].at[pl.ds(j * GSUB, GSUB)],
                sem_g[P],
            )
            for j in range(C // GSUB)
        ]
        return cps

    def _wb_cp(n, P):
        base = base0 + n * C
        return pltpu.make_async_copy(
            bufx[P], out_hbm.at[pl.ds(base * D, C * D)], sem_w[P])

    _load_idx(0, 0)
    for cp in _gather_cps(0):
        cp.start()

    def _step(ii, carry):
        for half in range(2):
            n = ii * 2 + half
            P = half
            for cp in _gather_cps(P):
                cp.wait()

            if half == 0:
                _load_idx(n + 1, 1 - P)
                for cp in _gather_cps(1 - P):
                    cp.start()
            else:
                @pl.when(ii < (NCHUNK // 2) - 1)
                def _prefetch():
                    _load_idx(n + 1, 1 - P)
                    for cp in _gather_cps(1 - P):
                        cp.start()

            @pl.when(ii > 0)
            def _drain():
                _wb_cp(n, P).wait()

            _compute(rows[P], prows[P], bufx[P])
            _wb_cp(n, P).start()
        return carry

    lax.fori_loop(0, NCHUNK // 2, _step, 0)
    _wb_cp(NCHUNK - 2, 0).wait()
    _wb_cp(NCHUNK - 1, 1).wait()


@jax.jit
def _emb(seq_flat, pid_flat, item_table, pos_table, ln_weight, ln_bias):
    mesh = plsc.VectorSubcoreMesh(core_axis_name="c", subcore_axis_name="s")
    f = functools.partial(
        pl.kernel,
        out_type=jax.ShapeDtypeStruct((SB * D,), jnp.float32),
        mesh=mesh,
        scratch_types=[
            pltpu.VMEM((C,), jnp.int32),
            pltpu.VMEM((C,), jnp.int32),
            pltpu.VMEM((C,), jnp.int32),
            pltpu.VMEM((C,), jnp.int32),
            pltpu.VMEM((C, D), jnp.float32),
            pltpu.VMEM((C, D), jnp.float32),
            pltpu.VMEM((C, D), jnp.float32),
            pltpu.VMEM((C, D), jnp.float32),
            pltpu.VMEM((C * D,), jnp.float32),
            pltpu.VMEM((C * D,), jnp.float32),
            pltpu.VMEM((D,), jnp.float32),
            pltpu.VMEM((D,), jnp.float32),
            pltpu.VMEM((32 * 17,), jnp.float32),
            pltpu.SemaphoreType.DMA,
            pltpu.SemaphoreType.DMA,
            pltpu.SemaphoreType.DMA,
            pltpu.SemaphoreType.DMA,
        ],
        compiler_params=pltpu.CompilerParams(
            needs_layout_passes=False, use_tc_tiling_on_sc=False),
    )(_body)
    return f(seq_flat, pid_flat, item_table, pos_table, ln_weight, ln_bias)


def kernel(input_sequence, position_ids, item_table, pos_table, ln_weight, ln_bias):
    seq_flat = input_sequence.reshape(SB)
    pid_flat = position_ids.reshape(SB)
    out = _emb(seq_flat, pid_flat, item_table, pos_table, ln_weight, ln_bias)
    return out.reshape(S, B, D)
